# Initial kernel scaffold; baseline (speedup 1.0000x reference)
import functools

import jax
import jax.numpy as jnp
from jax import lax
from jax.experimental import pallas as pl
from jax.experimental.pallas import tpu as pltpu
import jax.experimental.pallas.tpu_sc as plsc

N = 10000
E = 320000
D = 128
B = 1024
T = 4
C = 16

NW = 32
NSUB = 16
CHUNK = 128
NCHUNK = 80
EPAD = NW * NCHUNK * CHUNK
NP = N + 16
ROWS_PER_TILE = NP // NSUB
DEGW = 16

_mesh = plsc.VectorSubcoreMesh(core_axis_name="c", subcore_axis_name="s")


@functools.partial(
    pl.kernel,
    out_type=jax.ShapeDtypeStruct((2, NP, DEGW), jnp.float32),
    mesh=_mesh,
    scratch_types=[
        pltpu.VMEM((NCHUNK, CHUNK), jnp.int32),
        pltpu.VMEM((CHUNK, DEGW), jnp.float32),
        pltpu.VMEM_SHARED((NP, DEGW), jnp.float32),
    ],
)
def _sc_degree(dst_w, ones_hbm, zero_hbm, out, dst_v, ones_v, acc_sh):
    cid = lax.axis_index("c")
    sid = lax.axis_index("s")
    wid = cid * NSUB + sid
    r0 = sid * ROWS_PER_TILE
    pltpu.sync_copy(zero_hbm.at[pl.ds(r0, ROWS_PER_TILE)],
                    acc_sh.at[pl.ds(r0, ROWS_PER_TILE)])
    pltpu.sync_copy(ones_hbm, ones_v)
    pltpu.sync_copy(dst_w.at[wid], dst_v)
    plsc.subcore_barrier()

    def body(j, carry):
        pltpu.sync_copy(ones_v, acc_sh.at[dst_v.at[j]], add=True)
        return carry

    lax.fori_loop(0, NCHUNK, body, 0)
    plsc.subcore_barrier()
    pltpu.sync_copy(acc_sh.at[pl.ds(r0, ROWS_PER_TILE)],
                    out.at[cid, pl.ds(r0, ROWS_PER_TILE)])


@functools.partial(
    pl.kernel,
    out_type=jax.ShapeDtypeStruct((2, NP, D), jnp.float32),
    mesh=_mesh,
    scratch_types=[
        pltpu.VMEM((NCHUNK, CHUNK), jnp.int32),
        pltpu.VMEM((NCHUNK, CHUNK), jnp.int32),
        pltpu.VMEM((CHUNK, D), jnp.float32),
        pltpu.VMEM((CHUNK, D), jnp.float32),
        pltpu.VMEM_SHARED((NP, D), jnp.float32),
        pltpu.SemaphoreType.DMA,
        pltpu.SemaphoreType.DMA,
    ],
)
def _sc_segsum(hs, src_w, dst_w, zero_hbm, out,
               src_v, dst_v, rows0, rows1, acc_sh, sem0, sem1):
    cid = lax.axis_index("c")
    sid = lax.axis_index("s")
    wid = cid * NSUB + sid
    r0 = sid * ROWS_PER_TILE
    pltpu.sync_copy(zero_hbm.at[pl.ds(r0, ROWS_PER_TILE)],
                    acc_sh.at[pl.ds(r0, ROWS_PER_TILE)])
    pltpu.sync_copy(src_w.at[wid], src_v)
    pltpu.sync_copy(dst_w.at[wid], dst_v)
    plsc.subcore_barrier()

    pltpu.async_copy(hs.at[src_v.at[0]], rows0, sem0)

    def body(j, carry):
        @pl.when(j + 1 < NCHUNK)
        def _():
            @pl.when(lax.rem(j, 2) == 0)
            def _():
                pltpu.async_copy(hs.at[src_v.at[j + 1]], rows1, sem1)

            @pl.when(lax.rem(j, 2) == 1)
            def _():
                pltpu.async_copy(hs.at[src_v.at[j + 1]], rows0, sem0)

        @pl.when(lax.rem(j, 2) == 0)
        def _():
            pltpu.make_async_copy(hs.at[src_v.at[0]], rows0, sem0).wait()
            pltpu.sync_copy(rows0, acc_sh.at[dst_v.at[j]], add=True)

        @pl.when(lax.rem(j, 2) == 1)
        def _():
            pltpu.make_async_copy(hs.at[src_v.at[0]], rows1, sem1).wait()
            pltpu.sync_copy(rows1, acc_sh.at[dst_v.at[j]], add=True)

        return carry

    lax.fori_loop(0, NCHUNK, body, 0)
    plsc.subcore_barrier()
    pltpu.sync_copy(acc_sh.at[pl.ds(r0, ROWS_PER_TILE)],
                    out.at[cid, pl.ds(r0, ROWS_PER_TILE)])


_EGO_PER_W = B // NW


@functools.partial(
    pl.kernel,
    out_type=(
        jax.ShapeDtypeStruct((B, D), jnp.float32),
        jax.ShapeDtypeStruct((B, D), jnp.float32),
        jax.ShapeDtypeStruct((B, D), jnp.float32),
        jax.ShapeDtypeStruct((B, DEGW), jnp.float32),
        jax.ShapeDtypeStruct((B, DEGW), jnp.float32),
        jax.ShapeDtypeStruct((B, DEGW), jnp.int32),
    ),
    mesh=_mesh,
    scratch_types=[
        pltpu.VMEM((NW, _EGO_PER_W), jnp.int32),
        pltpu.VMEM((_EGO_PER_W, D), jnp.float32),
        pltpu.VMEM((_EGO_PER_W, D), jnp.float32),
        pltpu.VMEM((_EGO_PER_W, D), jnp.float32),
        pltpu.VMEM((_EGO_PER_W, DEGW), jnp.float32),
        pltpu.VMEM((_EGO_PER_W, DEGW), jnp.float32),
        pltpu.VMEM((_EGO_PER_W, DEGW), jnp.int32),
        pltpu.SemaphoreType.DMA,
    ],
)
def _sc_ego_gather(accA, accB, hs1, degA, degB, lab, ego_w,
                   a0_o, a1_o, h1_o, d0_o, d1_o, lb_o,
                   ego_v, bufA, bufB, bufH, bufDA, bufDB, bufL, sem):
    cid = lax.axis_index("c")
    sid = lax.axis_index("s")
    wid = cid * NSUB + sid
    pltpu.sync_copy(ego_w.at[wid], ego_v.at[wid])
    idx = ego_v.at[wid]
    pltpu.async_copy(accA.at[idx], bufA, sem).wait()
    pltpu.async_copy(accB.at[idx], bufB, sem).wait()
    pltpu.async_copy(hs1.at[idx], bufH, sem).wait()
    pltpu.async_copy(degA.at[idx], bufDA, sem).wait()
    pltpu.async_copy(degB.at[idx], bufDB, sem).wait()
    pltpu.async_copy(lab.at[idx], bufL, sem).wait()
    o0 = wid * _EGO_PER_W
    pltpu.sync_copy(bufA, a0_o.at[pl.ds(o0, _EGO_PER_W)])
    pltpu.sync_copy(bufB, a1_o.at[pl.ds(o0, _EGO_PER_W)])
    pltpu.sync_copy(bufH, h1_o.at[pl.ds(o0, _EGO_PER_W)])
    pltpu.sync_copy(bufDA, d0_o.at[pl.ds(o0, _EGO_PER_W)])
    pltpu.sync_copy(bufDB, d1_o.at[pl.ds(o0, _EGO_PER_W)])
    pltpu.sync_copy(bufL, lb_o.at[pl.ds(o0, _EGO_PER_W)])


def _tc_proj_body(x_ref, fcW_ref, fcb_ref, W0_ref, deg_ref, hs0_ref):
    x = x_ref[...]
    deg = deg_ref[0, :, 0:1] + deg_ref[1, :, 0:1] + 1.0
    dinv = lax.rsqrt(deg)
    rt = lax.broadcasted_iota(jnp.int32, (NP, D), 0) // (N // T)
    gh = jnp.zeros((NP, D), jnp.float32)
    for t in range(T):
        p = jnp.dot(x, fcW_ref[t], preferred_element_type=jnp.float32)
        p = p + fcb_ref[t][None, :]
        gh = jnp.where(rt == t, p, gh)
    hs0_ref[...] = jnp.dot(gh, W0_ref[...],
                           preferred_element_type=jnp.float32) * dinv


def _tc_layer_body(acc_ref, hs_ref, deg_ref, b_ref, W_ref, out_ref):
    deg = deg_ref[0, :, 0:1] + deg_ref[1, :, 0:1] + 1.0
    dinv = lax.rsqrt(deg)
    g = dinv * (acc_ref[0] + acc_ref[1] + hs_ref[...]) + b_ref[...]
    g = jnp.maximum(g, 0.0)
    out_ref[...] = jnp.dot(g, W_ref[...],
                           preferred_element_type=jnp.float32) * dinv


def _tc_final_body(a0_ref, a1_ref, h1_ref, d0_ref, d1_ref, b_ref,
                   pW_ref, pb_ref, out_ref):
    deg = d0_ref[:, 0:1] + d1_ref[:, 0:1] + 1.0
    dinv = lax.rsqrt(deg)
    g = dinv * (a0_ref[...] + a1_ref[...] + h1_ref[...]) + b_ref[...]
    g = jnp.maximum(g, 0.0)
    out_ref[...] = jnp.dot(g, pW_ref[...],
                           preferred_element_type=jnp.float32) + pb_ref[...]


def kernel(x, label, seqs, edge_index, node_type, fcW, fcb, gcnW, gcnb,
           reW, re_wtype, re_b, predW, predb):
    f32 = jnp.float32
    src = edge_index[0].astype(jnp.int32)
    dst = edge_index[1].astype(jnp.int32)
    padlen = EPAD - E
    src_w = jnp.concatenate([src, jnp.full((padlen,), N, jnp.int32)]
                            ).reshape(NW, NCHUNK, CHUNK)
    dst_w = jnp.concatenate([dst, jnp.full((padlen,), N, jnp.int32)]
                            ).reshape(NW, NCHUNK, CHUNK)
    x_pad = jnp.pad(x, ((0, NP - N), (0, 0)))
    ones16 = jnp.ones((CHUNK, DEGW), f32)
    zdeg = jnp.zeros((NP, DEGW), f32)
    zacc = jnp.zeros((NP, D), f32)
    lab16 = jnp.broadcast_to(
        jnp.pad(label.astype(jnp.int32), (0, NP - N))[:, None], (NP, DEGW))
    ego = seqs[:, 0].astype(jnp.int32)
    ego_w = ego.reshape(NW, _EGO_PER_W)
    predW_pad = jnp.pad(predW, ((0, 0), (0, D - C)))
    predb_pad = jnp.pad(predb, (0, D - C)).reshape(1, D)

    deg2 = _sc_degree(dst_w, ones16, zdeg)

    hs0 = pl.pallas_call(
        _tc_proj_body,
        out_shape=jax.ShapeDtypeStruct((NP, D), f32),
    )(x_pad, fcW, fcb, gcnW[0], deg2)

    acc1 = _sc_segsum(hs0, src_w, dst_w, zacc)
    hs1 = pl.pallas_call(
        _tc_layer_body,
        out_shape=jax.ShapeDtypeStruct((NP, D), f32),
    )(acc1, hs0, deg2, gcnb[0].reshape(1, D), gcnW[1])
    acc2 = _sc_segsum(hs1, src_w, dst_w, zacc)

    a0, a1, h1, d0, d1, lb = _sc_ego_gather(
        acc2[0], acc2[1], hs1, deg2[0], deg2[1], lab16, ego_w)

    out = pl.pallas_call(
        _tc_final_body,
        out_shape=jax.ShapeDtypeStruct((B, D), f32),
    )(a0, a1, h1, d0, d1, gcnb[1].reshape(1, D), predW_pad, predb_pad)

    return (out[:, :C], lb[:, 0])


# speedup vs baseline: 8.1307x; 8.1307x over previous
import functools

import jax
import jax.numpy as jnp
from jax import lax
from jax.experimental import pallas as pl
from jax.experimental.pallas import tpu as pltpu
import jax.experimental.pallas.tpu_sc as plsc

N = 10000
E = 320000
D = 128
B = 1024
T = 4
C = 16

NW = 32
NSUB = 16
CHUNK = 128
NCHUNK = 80
EPAD = NW * NCHUNK * CHUNK
NP = 10112
ROWS_PER_TILE = NP // NSUB
DEGW = 16

_mesh = plsc.VectorSubcoreMesh(core_axis_name="c", subcore_axis_name="s")


@functools.partial(
    pl.kernel,
    out_type=jax.ShapeDtypeStruct((2, NP, D), jnp.float32),
    mesh=_mesh,
    scratch_types=[
        pltpu.VMEM((NCHUNK, CHUNK), jnp.int32),
        pltpu.VMEM((CHUNK, D), jnp.float32),
        pltpu.VMEM_SHARED((NP, D), jnp.float32),
    ],
)
def _sc_degree(dst_w, ones_hbm, zero_hbm, out, dst_v, ones_v, acc_sh):
    cid = lax.axis_index("c")
    sid = lax.axis_index("s")
    wid = cid * NSUB + sid
    r0 = sid * ROWS_PER_TILE
    pltpu.sync_copy(zero_hbm.at[pl.ds(r0, ROWS_PER_TILE)],
                    acc_sh.at[pl.ds(r0, ROWS_PER_TILE)])
    pltpu.sync_copy(ones_hbm, ones_v)
    pltpu.sync_copy(dst_w.at[wid], dst_v)
    plsc.subcore_barrier()

    def body(j, carry):
        pltpu.sync_copy(ones_v, acc_sh.at[dst_v.at[j]], add=True)
        return carry

    lax.fori_loop(0, NCHUNK, body, 0)
    plsc.subcore_barrier()
    pltpu.sync_copy(acc_sh.at[pl.ds(r0, ROWS_PER_TILE)],
                    out.at[cid, pl.ds(r0, ROWS_PER_TILE)])


@functools.partial(
    pl.kernel,
    out_type=jax.ShapeDtypeStruct((2, NP, D), jnp.float32),
    mesh=_mesh,
    scratch_types=[
        pltpu.VMEM((NCHUNK, CHUNK), jnp.int32),
        pltpu.VMEM((NCHUNK, CHUNK), jnp.int32),
        pltpu.VMEM((CHUNK, D), jnp.float32),
        pltpu.VMEM_SHARED((NP, D), jnp.float32),
        pltpu.SemaphoreType.DMA,
    ],
)
def _sc_segsum(hs, src_w, dst_w, zero_hbm, out,
               src_v, dst_v, rows0, acc_sh, sem0):
    cid = lax.axis_index("c")
    sid = lax.axis_index("s")
    wid = cid * NSUB + sid
    r0 = sid * ROWS_PER_TILE
    pltpu.sync_copy(zero_hbm.at[pl.ds(r0, ROWS_PER_TILE)],
                    acc_sh.at[pl.ds(r0, ROWS_PER_TILE)])
    pltpu.sync_copy(src_w.at[wid], src_v)
    pltpu.sync_copy(dst_w.at[wid], dst_v)
    plsc.subcore_barrier()

    def body(j, carry):
        pltpu.async_copy(hs.at[src_v.at[j]], rows0, sem0).wait()
        pltpu.sync_copy(rows0, acc_sh.at[dst_v.at[j]], add=True)
        return carry

    lax.fori_loop(0, NCHUNK, body, 0)
    plsc.subcore_barrier()
    pltpu.sync_copy(acc_sh.at[pl.ds(r0, ROWS_PER_TILE)],
                    out.at[cid, pl.ds(r0, ROWS_PER_TILE)])


_EGO_PER_W = B // NW


@functools.partial(
    pl.kernel,
    out_type=(
        jax.ShapeDtypeStruct((B, D), jnp.float32),
        jax.ShapeDtypeStruct((B, D), jnp.float32),
        jax.ShapeDtypeStruct((B, D), jnp.float32),
        jax.ShapeDtypeStruct((B, D), jnp.float32),
    ),
    mesh=_mesh,
    scratch_types=[
        pltpu.VMEM((NW, _EGO_PER_W), jnp.int32),
        pltpu.VMEM((_EGO_PER_W, D), jnp.float32),
        pltpu.VMEM((_EGO_PER_W, D), jnp.float32),
        pltpu.VMEM((_EGO_PER_W, D), jnp.float32),
        pltpu.VMEM((_EGO_PER_W, D), jnp.float32),
        pltpu.SemaphoreType.DMA,
    ],
)
def _sc_ego_gather(accA, accB, hs1, misc, ego_w,
                   a0_o, a1_o, h1_o, mg_o,
                   ego_v, bufA, bufB, bufH, bufM, sem):
    cid = lax.axis_index("c")
    sid = lax.axis_index("s")
    wid = cid * NSUB + sid
    pltpu.sync_copy(ego_w.at[wid], ego_v.at[wid])
    idx = ego_v.at[wid]
    pltpu.async_copy(accA.at[idx], bufA, sem).wait()
    pltpu.async_copy(accB.at[idx], bufB, sem).wait()
    pltpu.async_copy(hs1.at[idx], bufH, sem).wait()
    pltpu.async_copy(misc.at[idx], bufM, sem).wait()
    o0 = wid * _EGO_PER_W
    pltpu.sync_copy(bufA, a0_o.at[pl.ds(o0, _EGO_PER_W)])
    pltpu.sync_copy(bufB, a1_o.at[pl.ds(o0, _EGO_PER_W)])
    pltpu.sync_copy(bufH, h1_o.at[pl.ds(o0, _EGO_PER_W)])
    pltpu.sync_copy(bufM, mg_o.at[pl.ds(o0, _EGO_PER_W)])


def _tc_proj_body(x_ref, fcW_ref, fcb_ref, W0_ref, deg_ref, hs0_ref):
    x = x_ref[...]
    deg = deg_ref[0, :, 0:1] + deg_ref[1, :, 0:1] + 1.0
    dinv = lax.rsqrt(deg)
    rt = lax.broadcasted_iota(jnp.int32, (NP, D), 0) // (N // T)
    gh = jnp.zeros((NP, D), jnp.float32)
    for t in range(T):
        p = jnp.dot(x, fcW_ref[t], preferred_element_type=jnp.float32)
        p = p + fcb_ref[t]
        gh = jnp.where(rt == t, p, gh)
    hs0_ref[...] = jnp.dot(gh, W0_ref[...],
                           preferred_element_type=jnp.float32) * dinv


def _tc_layer_body(acc_ref, hs_ref, deg_ref, b_ref, W_ref, lab_ref,
                   out_ref, misc_ref):
    deg = deg_ref[0, :, 0:1] + deg_ref[1, :, 0:1] + 1.0
    dinv = lax.rsqrt(deg)
    g = dinv * (acc_ref[0] + acc_ref[1] + hs_ref[...]) + b_ref[...]
    g = jnp.maximum(g, 0.0)
    out_ref[...] = jnp.dot(g, W_ref[...],
                           preferred_element_type=jnp.float32) * dinv
    col = lax.broadcasted_iota(jnp.int32, (NP, D), 1)
    misc_ref[...] = jnp.where(col == 0, dinv, 0.0) + jnp.where(
        col == 1, lab_ref[:, 0:1], 0.0)


def _tc_final_body(a0_ref, a1_ref, h1_ref, mg_ref, b_ref,
                   pW_ref, pb_ref, out_ref):
    dinv = mg_ref[:, 0:1]
    g = dinv * (a0_ref[...] + a1_ref[...] + h1_ref[...]) + b_ref[...]
    g = jnp.maximum(g, 0.0)
    out_ref[...] = jnp.dot(g, pW_ref[...],
                           preferred_element_type=jnp.float32) + pb_ref[...]


def kernel(x, label, seqs, edge_index, node_type, fcW, fcb, gcnW, gcnb,
           reW, re_wtype, re_b, predW, predb):
    f32 = jnp.float32
    src = edge_index[0].astype(jnp.int32)
    dst = edge_index[1].astype(jnp.int32)
    padlen = EPAD - E
    src_w = jnp.concatenate([src, jnp.full((padlen,), N, jnp.int32)]
                            ).reshape(NW, NCHUNK, CHUNK)
    dst_w = jnp.concatenate([dst, jnp.full((padlen,), N, jnp.int32)]
                            ).reshape(NW, NCHUNK, CHUNK)
    x_pad = jnp.pad(x, ((0, NP - N), (0, 0)))
    ones128 = jnp.ones((CHUNK, D), f32)
    zacc = jnp.zeros((NP, D), f32)
    labf = jnp.broadcast_to(
        jnp.pad(label.astype(f32), (0, NP - N))[:, None], (NP, 8))
    ego = seqs[:, 0].astype(jnp.int32)
    ego_w = ego.reshape(NW, _EGO_PER_W)
    predW_pad = jnp.pad(predW, ((0, 0), (0, D - C)))
    predb_pad = jnp.pad(predb, (0, D - C)).reshape(1, D)

    deg2 = _sc_degree(dst_w, ones128, zacc)

    hs0 = pl.pallas_call(
        _tc_proj_body,
        out_shape=jax.ShapeDtypeStruct((NP, D), f32),
    )(x_pad, fcW, fcb.reshape(T, 1, D), gcnW[0], deg2)

    acc1 = _sc_segsum(hs0, src_w, dst_w, zacc)
    hs1, misc = pl.pallas_call(
        _tc_layer_body,
        out_shape=(jax.ShapeDtypeStruct((NP, D), f32),
                   jax.ShapeDtypeStruct((NP, D), f32)),
    )(acc1, hs0, deg2, gcnb[0].reshape(1, D), gcnW[1], labf)
    acc2 = _sc_segsum(hs1, src_w, dst_w, zacc)

    a0, a1, h1, mg = _sc_ego_gather(acc2[0], acc2[1], hs1, misc, ego_w)

    out = pl.pallas_call(
        _tc_final_body,
        out_shape=jax.ShapeDtypeStruct((B, D), f32),
    )(a0, a1, h1, mg, gcnb[1].reshape(1, D), predW_pad, predb_pad)

    return (out[:, :C], mg[:, 1].astype(label.dtype))


# speedup vs baseline: 9.1639x; 1.1271x over previous
import functools

import jax
import jax.numpy as jnp
from jax import lax
from jax.experimental import pallas as pl
from jax.experimental.pallas import tpu as pltpu
import jax.experimental.pallas.tpu_sc as plsc

N = 10000
E = 320000
D = 128
B = 1024
T = 4
C = 16

NW = 32
NSUB = 16
CHUNK = 128
NCHUNK = 80
QCHUNK = 16
EPAD = NW * NCHUNK * CHUNK
NP = 10112
ROWS_PER_TILE = NP // NSUB
DEGW = 16

_mesh = plsc.VectorSubcoreMesh(core_axis_name="c", subcore_axis_name="s")


@functools.partial(
    pl.kernel,
    out_type=jax.ShapeDtypeStruct((2, NP, D), jnp.float32),
    mesh=_mesh,
    scratch_types=[
        pltpu.VMEM((NCHUNK, CHUNK), jnp.int32),
        pltpu.VMEM((CHUNK, D), jnp.float32),
        pltpu.VMEM_SHARED((NP, D), jnp.float32),
    ],
)
def _sc_degree(dst_w, ones_hbm, zero_hbm, out, dst_v, ones_v, acc_sh):
    cid = lax.axis_index("c")
    sid = lax.axis_index("s")
    wid = cid * NSUB + sid
    r0 = sid * ROWS_PER_TILE
    pltpu.sync_copy(zero_hbm.at[pl.ds(r0, ROWS_PER_TILE)],
                    acc_sh.at[pl.ds(r0, ROWS_PER_TILE)])
    pltpu.sync_copy(ones_hbm, ones_v)
    pltpu.sync_copy(dst_w.at[wid], dst_v)
    plsc.subcore_barrier()

    def body(j, carry):
        pltpu.sync_copy(ones_v, acc_sh.at[dst_v.at[j]], add=True)
        return carry

    lax.fori_loop(0, NCHUNK, body, 0)
    plsc.subcore_barrier()
    pltpu.sync_copy(acc_sh.at[pl.ds(r0, ROWS_PER_TILE)],
                    out.at[cid, pl.ds(r0, ROWS_PER_TILE)])


@functools.partial(
    pl.kernel,
    out_type=jax.ShapeDtypeStruct((2, NP, D), jnp.float32),
    mesh=_mesh,
    scratch_types=[
        pltpu.VMEM((2, QCHUNK, CHUNK), jnp.int32),
        pltpu.VMEM((2, QCHUNK, CHUNK), jnp.int32),
        pltpu.VMEM((CHUNK, D), jnp.float32),
        pltpu.VMEM((CHUNK, D), jnp.float32),
        pltpu.VMEM_SHARED((NP, D), jnp.float32),
        pltpu.SemaphoreType.DMA,
        pltpu.SemaphoreType.DMA,
        pltpu.SemaphoreType.DMA,
    ],
)
def _sc_segsum(hs, src_w, dst_w, zero_hbm, out,
               src_v, dst_v, rows0, rows1, acc_sh, sem0, sem1, isem):
    cid = lax.axis_index("c")
    sid = lax.axis_index("s")
    wid = cid * NSUB + sid
    r0 = sid * ROWS_PER_TILE
    pltpu.sync_copy(zero_hbm.at[pl.ds(r0, ROWS_PER_TILE)],
                    acc_sh.at[pl.ds(r0, ROWS_PER_TILE)])
    pltpu.sync_copy(src_w.at[wid, pl.ds(0, QCHUNK)], src_v.at[0])
    pltpu.sync_copy(dst_w.at[wid, pl.ds(0, QCHUNK)], dst_v.at[0])
    pltpu.async_copy(src_w.at[wid, pl.ds(QCHUNK, QCHUNK)], src_v.at[1], isem)
    pltpu.async_copy(dst_w.at[wid, pl.ds(QCHUNK, QCHUNK)], dst_v.at[1], isem)
    plsc.subcore_barrier()

    pltpu.async_copy(hs.at[src_v.at[0, 0]], rows0, sem0)

    def body(j, carry):
        q = j // QCHUNK
        k = lax.rem(j, QCHUNK)
        qb = lax.rem(q, 2)

        @pl.when(k == QCHUNK - 1)
        def _():
            @pl.when(j + 1 < NCHUNK)
            def _():
                pltpu.make_async_copy(
                    src_w.at[wid, pl.ds(0, QCHUNK)], src_v.at[qb], isem).wait()
                pltpu.make_async_copy(
                    dst_w.at[wid, pl.ds(0, QCHUNK)], dst_v.at[qb], isem).wait()

        @pl.when(j + 1 < NCHUNK)
        def _():
            jn = j + 1
            qn = lax.rem(jn // QCHUNK, 2)
            kn = lax.rem(jn, QCHUNK)

            @pl.when(lax.rem(jn, 2) == 0)
            def _():
                pltpu.async_copy(hs.at[src_v.at[qn, kn]], rows0, sem0)

            @pl.when(lax.rem(jn, 2) == 1)
            def _():
                pltpu.async_copy(hs.at[src_v.at[qn, kn]], rows1, sem1)

        @pl.when(lax.rem(j, 2) == 0)
        def _():
            pltpu.make_async_copy(hs.at[src_v.at[0, 0]], rows0, sem0).wait()
            pltpu.sync_copy(rows0, acc_sh.at[dst_v.at[qb, k]], add=True)

        @pl.when(lax.rem(j, 2) == 1)
        def _():
            pltpu.make_async_copy(hs.at[src_v.at[0, 0]], rows1, sem1).wait()
            pltpu.sync_copy(rows1, acc_sh.at[dst_v.at[qb, k]], add=True)

        @pl.when((k == QCHUNK - 1) & (j + QCHUNK + 1 < NCHUNK))
        def _():
            off = (j + QCHUNK + 1) // QCHUNK * QCHUNK
            pltpu.async_copy(src_w.at[wid, pl.ds(off, QCHUNK)],
                             src_v.at[qb], isem)
            pltpu.async_copy(dst_w.at[wid, pl.ds(off, QCHUNK)],
                             dst_v.at[qb], isem)

        return carry

    lax.fori_loop(0, NCHUNK, body, 0)
    plsc.subcore_barrier()
    pltpu.sync_copy(acc_sh.at[pl.ds(r0, ROWS_PER_TILE)],
                    out.at[cid, pl.ds(r0, ROWS_PER_TILE)])


_EGO_PER_W = B // NW


@functools.partial(
    pl.kernel,
    out_type=(
        jax.ShapeDtypeStruct((B, D), jnp.float32),
        jax.ShapeDtypeStruct((B, D), jnp.float32),
        jax.ShapeDtypeStruct((B, D), jnp.float32),
        jax.ShapeDtypeStruct((B, D), jnp.float32),
    ),
    mesh=_mesh,
    scratch_types=[
        pltpu.VMEM((NW, _EGO_PER_W), jnp.int32),
        pltpu.VMEM((_EGO_PER_W, D), jnp.float32),
        pltpu.VMEM((_EGO_PER_W, D), jnp.float32),
        pltpu.VMEM((_EGO_PER_W, D), jnp.float32),
        pltpu.VMEM((_EGO_PER_W, D), jnp.float32),
        pltpu.SemaphoreType.DMA,
    ],
)
def _sc_ego_gather(accA, accB, hs1, misc, ego_w,
                   a0_o, a1_o, h1_o, mg_o,
                   ego_v, bufA, bufB, bufH, bufM, sem):
    cid = lax.axis_index("c")
    sid = lax.axis_index("s")
    wid = cid * NSUB + sid
    pltpu.sync_copy(ego_w.at[wid], ego_v.at[wid])
    idx = ego_v.at[wid]
    pltpu.async_copy(accA.at[idx], bufA, sem).wait()
    pltpu.async_copy(accB.at[idx], bufB, sem).wait()
    pltpu.async_copy(hs1.at[idx], bufH, sem).wait()
    pltpu.async_copy(misc.at[idx], bufM, sem).wait()
    o0 = wid * _EGO_PER_W
    pltpu.sync_copy(bufA, a0_o.at[pl.ds(o0, _EGO_PER_W)])
    pltpu.sync_copy(bufB, a1_o.at[pl.ds(o0, _EGO_PER_W)])
    pltpu.sync_copy(bufH, h1_o.at[pl.ds(o0, _EGO_PER_W)])
    pltpu.sync_copy(bufM, mg_o.at[pl.ds(o0, _EGO_PER_W)])


def _tc_proj_body(x_ref, fcW_ref, fcb_ref, W0_ref, deg_ref, hs0_ref):
    x = x_ref[...]
    deg = deg_ref[0, :, 0:1] + deg_ref[1, :, 0:1] + 1.0
    dinv = lax.rsqrt(deg)
    rt = lax.broadcasted_iota(jnp.int32, (NP, D), 0) // (N // T)
    gh = jnp.zeros((NP, D), jnp.float32)
    for t in range(T):
        p = jnp.dot(x, fcW_ref[t], preferred_element_type=jnp.float32)
        p = p + fcb_ref[t]
        gh = jnp.where(rt == t, p, gh)
    hs0_ref[...] = jnp.dot(gh, W0_ref[...],
                           preferred_element_type=jnp.float32) * dinv


def _tc_layer_body(acc_ref, hs_ref, deg_ref, b_ref, W_ref, lab_ref,
                   out_ref, misc_ref):
    deg = deg_ref[0, :, 0:1] + deg_ref[1, :, 0:1] + 1.0
    dinv = lax.rsqrt(deg)
    g = dinv * (acc_ref[0] + acc_ref[1] + hs_ref[...]) + b_ref[...]
    g = jnp.maximum(g, 0.0)
    out_ref[...] = jnp.dot(g, W_ref[...],
                           preferred_element_type=jnp.float32) * dinv
    col = lax.broadcasted_iota(jnp.int32, (NP, D), 1)
    misc_ref[...] = jnp.where(col == 0, dinv, 0.0) + jnp.where(
        col == 1, lab_ref[:, 0:1], 0.0)


def _tc_final_body(a0_ref, a1_ref, h1_ref, mg_ref, b_ref,
                   pW_ref, pb_ref, out_ref):
    dinv = mg_ref[:, 0:1]
    g = dinv * (a0_ref[...] + a1_ref[...] + h1_ref[...]) + b_ref[...]
    g = jnp.maximum(g, 0.0)
    out_ref[...] = jnp.dot(g, pW_ref[...],
                           preferred_element_type=jnp.float32) + pb_ref[...]


def kernel(x, label, seqs, edge_index, node_type, fcW, fcb, gcnW, gcnb,
           reW, re_wtype, re_b, predW, predb):
    f32 = jnp.float32
    src = edge_index[0].astype(jnp.int32)
    dst = edge_index[1].astype(jnp.int32)
    padlen = EPAD - E
    src_w = jnp.concatenate([src, jnp.full((padlen,), N, jnp.int32)]
                            ).reshape(NW, NCHUNK, CHUNK)
    dst_w = jnp.concatenate([dst, jnp.full((padlen,), N, jnp.int32)]
                            ).reshape(NW, NCHUNK, CHUNK)
    x_pad = jnp.pad(x, ((0, NP - N), (0, 0)))
    ones128 = jnp.ones((CHUNK, D), f32)
    zacc = jnp.zeros((NP, D), f32)
    labf = jnp.broadcast_to(
        jnp.pad(label.astype(f32), (0, NP - N))[:, None], (NP, 8))
    ego = seqs[:, 0].astype(jnp.int32)
    ego_w = ego.reshape(NW, _EGO_PER_W)
    predW_pad = jnp.pad(predW, ((0, 0), (0, D - C)))
    predb_pad = jnp.pad(predb, (0, D - C)).reshape(1, D)

    deg2 = _sc_degree(dst_w, ones128, zacc)

    hs0 = pl.pallas_call(
        _tc_proj_body,
        out_shape=jax.ShapeDtypeStruct((NP, D), f32),
    )(x_pad, fcW, fcb.reshape(T, 1, D), gcnW[0], deg2)

    acc1 = _sc_segsum(hs0, src_w, dst_w, zacc)
    hs1, misc = pl.pallas_call(
        _tc_layer_body,
        out_shape=(jax.ShapeDtypeStruct((NP, D), f32),
                   jax.ShapeDtypeStruct((NP, D), f32)),
    )(acc1, hs0, deg2, gcnb[0].reshape(1, D), gcnW[1], labf)
    acc2 = _sc_segsum(hs1, src_w, dst_w, zacc)

    a0, a1, h1, mg = _sc_ego_gather(acc2[0], acc2[1], hs1, misc, ego_w)

    out = pl.pallas_call(
        _tc_final_body,
        out_shape=jax.ShapeDtypeStruct((B, D), f32),
    )(a0, a1, h1, mg, gcnb[1].reshape(1, D), predW_pad, predb_pad)

    return (out[:, :C], mg[:, 1].astype(label.dtype))


# speedup vs baseline: 9.5425x; 1.0413x over previous
import functools

import jax
import jax.numpy as jnp
from jax import lax
from jax.experimental import pallas as pl
from jax.experimental.pallas import tpu as pltpu
import jax.experimental.pallas.tpu_sc as plsc

N = 10000
E = 320000
D = 128
B = 1024
T = 4
C = 16

NW = 32
NSUB = 16
CHUNK = 128
NCHUNK = 80
QCHUNK = 16
EPAD = NW * NCHUNK * CHUNK
NP = 10112
ROWS_PER_TILE = NP // NSUB
DEGW = 16

_mesh = plsc.VectorSubcoreMesh(core_axis_name="c", subcore_axis_name="s")


@functools.partial(
    pl.kernel,
    out_type=jax.ShapeDtypeStruct((2, NP, D), jnp.float32),
    mesh=_mesh,
    scratch_types=[
        pltpu.VMEM((NCHUNK, CHUNK), jnp.int32),
        pltpu.VMEM((CHUNK, D), jnp.float32),
        pltpu.VMEM_SHARED((NP, D), jnp.float32),
    ],
)
def _sc_degree(dst_w, ones_hbm, zero_hbm, out, dst_v, ones_v, acc_sh):
    cid = lax.axis_index("c")
    sid = lax.axis_index("s")
    wid = cid * NSUB + sid
    r0 = sid * ROWS_PER_TILE
    pltpu.sync_copy(zero_hbm.at[pl.ds(r0, ROWS_PER_TILE)],
                    acc_sh.at[pl.ds(r0, ROWS_PER_TILE)])
    pltpu.sync_copy(ones_hbm, ones_v)
    pltpu.sync_copy(dst_w.at[wid], dst_v)
    plsc.subcore_barrier()

    def body(j, carry):
        pltpu.sync_copy(ones_v, acc_sh.at[dst_v.at[j]], add=True)
        return carry

    lax.fori_loop(0, NCHUNK, body, 0)
    plsc.subcore_barrier()
    pltpu.sync_copy(acc_sh.at[pl.ds(r0, ROWS_PER_TILE)],
                    out.at[cid, pl.ds(r0, ROWS_PER_TILE)])


NC_A = 112
NC_B = 48
TOTC = NSUB * (NC_A + NC_B)


@functools.partial(
    pl.kernel,
    out_type=jax.ShapeDtypeStruct((2, NP, D), jnp.float32),
    mesh=_mesh,
    scratch_types=[
        pltpu.VMEM((2, QCHUNK, CHUNK), jnp.int32),
        pltpu.VMEM((2, QCHUNK, CHUNK), jnp.int32),
        pltpu.VMEM((CHUNK, D), jnp.float32),
        pltpu.VMEM((CHUNK, D), jnp.float32),
        pltpu.VMEM_SHARED((NP, D), jnp.float32),
        pltpu.SemaphoreType.DMA,
        pltpu.SemaphoreType.DMA,
        pltpu.SemaphoreType.DMA,
    ],
)
def _sc_segsum(hs, src_w, dst_w, zero_hbm, out,
               src_v, dst_v, rows0, rows1, acc_sh, sem0, sem1, isem):
    cid = lax.axis_index("c")
    sid = lax.axis_index("s")
    r0 = sid * ROWS_PER_TILE
    nchunk = lax.select(cid == 0, NC_A, NC_B)
    base = lax.select(cid == 0, sid * NC_A, NSUB * NC_A + sid * NC_B)
    pltpu.sync_copy(zero_hbm.at[pl.ds(r0, ROWS_PER_TILE)],
                    acc_sh.at[pl.ds(r0, ROWS_PER_TILE)])
    pltpu.sync_copy(src_w.at[pl.ds(base, QCHUNK)], src_v.at[0])
    pltpu.sync_copy(dst_w.at[pl.ds(base, QCHUNK)], dst_v.at[0])
    pltpu.async_copy(src_w.at[pl.ds(base + QCHUNK, QCHUNK)], src_v.at[1], isem)
    pltpu.async_copy(dst_w.at[pl.ds(base + QCHUNK, QCHUNK)], dst_v.at[1], isem)
    plsc.subcore_barrier()

    pltpu.async_copy(hs.at[src_v.at[0, 0]], rows0, sem0)

    def body(j, carry):
        q = j // QCHUNK
        k = lax.rem(j, QCHUNK)
        qb = lax.rem(q, 2)

        @pl.when(k == QCHUNK - 1)
        def _():
            @pl.when(j + 1 < nchunk)
            def _():
                pltpu.make_async_copy(
                    src_w.at[pl.ds(base, QCHUNK)], src_v.at[qb], isem).wait()
                pltpu.make_async_copy(
                    dst_w.at[pl.ds(base, QCHUNK)], dst_v.at[qb], isem).wait()

        @pl.when(j + 1 < nchunk)
        def _():
            jn = j + 1
            qn = lax.rem(jn // QCHUNK, 2)
            kn = lax.rem(jn, QCHUNK)

            @pl.when(lax.rem(jn, 2) == 0)
            def _():
                pltpu.async_copy(hs.at[src_v.at[qn, kn]], rows0, sem0)

            @pl.when(lax.rem(jn, 2) == 1)
            def _():
                pltpu.async_copy(hs.at[src_v.at[qn, kn]], rows1, sem1)

        @pl.when(lax.rem(j, 2) == 0)
        def _():
            pltpu.make_async_copy(hs.at[src_v.at[0, 0]], rows0, sem0).wait()
            pltpu.sync_copy(rows0, acc_sh.at[dst_v.at[qb, k]], add=True)

        @pl.when(lax.rem(j, 2) == 1)
        def _():
            pltpu.make_async_copy(hs.at[src_v.at[0, 0]], rows1, sem1).wait()
            pltpu.sync_copy(rows1, acc_sh.at[dst_v.at[qb, k]], add=True)

        @pl.when((k == QCHUNK - 1) & (j + QCHUNK + 1 < nchunk))
        def _():
            off = (j + QCHUNK + 1) // QCHUNK * QCHUNK
            pltpu.async_copy(src_w.at[pl.ds(base + off, QCHUNK)],
                             src_v.at[qb], isem)
            pltpu.async_copy(dst_w.at[pl.ds(base + off, QCHUNK)],
                             dst_v.at[qb], isem)

        return carry

    lax.fori_loop(0, nchunk, body, 0)
    plsc.subcore_barrier()
    pltpu.sync_copy(acc_sh.at[pl.ds(r0, ROWS_PER_TILE)],
                    out.at[cid, pl.ds(r0, ROWS_PER_TILE)])


_EGO_PER_W = B // NW


@functools.partial(
    pl.kernel,
    out_type=(
        jax.ShapeDtypeStruct((B, D), jnp.float32),
        jax.ShapeDtypeStruct((B, D), jnp.float32),
        jax.ShapeDtypeStruct((B, D), jnp.float32),
        jax.ShapeDtypeStruct((B, D), jnp.float32),
    ),
    mesh=_mesh,
    scratch_types=[
        pltpu.VMEM((NW, _EGO_PER_W), jnp.int32),
        pltpu.VMEM((_EGO_PER_W, D), jnp.float32),
        pltpu.VMEM((_EGO_PER_W, D), jnp.float32),
        pltpu.VMEM((_EGO_PER_W, D), jnp.float32),
        pltpu.VMEM((_EGO_PER_W, D), jnp.float32),
        pltpu.SemaphoreType.DMA,
    ],
)
def _sc_ego_gather(accA, accB, hs1, misc, ego_w,
                   a0_o, a1_o, h1_o, mg_o,
                   ego_v, bufA, bufB, bufH, bufM, sem):
    cid = lax.axis_index("c")
    sid = lax.axis_index("s")
    wid = cid * NSUB + sid
    pltpu.sync_copy(ego_w.at[wid], ego_v.at[wid])
    idx = ego_v.at[wid]
    pltpu.async_copy(accA.at[idx], bufA, sem).wait()
    pltpu.async_copy(accB.at[idx], bufB, sem).wait()
    pltpu.async_copy(hs1.at[idx], bufH, sem).wait()
    pltpu.async_copy(misc.at[idx], bufM, sem).wait()
    o0 = wid * _EGO_PER_W
    pltpu.sync_copy(bufA, a0_o.at[pl.ds(o0, _EGO_PER_W)])
    pltpu.sync_copy(bufB, a1_o.at[pl.ds(o0, _EGO_PER_W)])
    pltpu.sync_copy(bufH, h1_o.at[pl.ds(o0, _EGO_PER_W)])
    pltpu.sync_copy(bufM, mg_o.at[pl.ds(o0, _EGO_PER_W)])


def _tc_proj_body(x_ref, fcW_ref, fcb_ref, W0_ref, deg_ref, hs0_ref):
    x = x_ref[...]
    deg = deg_ref[0, :, 0:1] + deg_ref[1, :, 0:1] + 1.0
    dinv = lax.rsqrt(deg)
    rt = lax.broadcasted_iota(jnp.int32, (NP, D), 0) // (N // T)
    gh = jnp.zeros((NP, D), jnp.float32)
    for t in range(T):
        p = jnp.dot(x, fcW_ref[t], preferred_element_type=jnp.float32)
        p = p + fcb_ref[t]
        gh = jnp.where(rt == t, p, gh)
    hs0_ref[...] = jnp.dot(gh, W0_ref[...],
                           preferred_element_type=jnp.float32) * dinv


def _tc_layer_body(acc_ref, hs_ref, deg_ref, b_ref, W_ref, lab_ref,
                   out_ref, misc_ref):
    deg = deg_ref[0, :, 0:1] + deg_ref[1, :, 0:1] + 1.0
    dinv = lax.rsqrt(deg)
    g = dinv * (acc_ref[0] + acc_ref[1] + hs_ref[...]) + b_ref[...]
    g = jnp.maximum(g, 0.0)
    out_ref[...] = jnp.dot(g, W_ref[...],
                           preferred_element_type=jnp.float32) * dinv
    col = lax.broadcasted_iota(jnp.int32, (NP, D), 1)
    misc_ref[...] = jnp.where(col == 0, dinv, 0.0) + jnp.where(
        col == 1, lab_ref[:, 0:1], 0.0)


def _tc_final_body(a0_ref, a1_ref, h1_ref, mg_ref, b_ref,
                   pW_ref, pb_ref, out_ref):
    dinv = mg_ref[:, 0:1]
    g = dinv * (a0_ref[...] + a1_ref[...] + h1_ref[...]) + b_ref[...]
    g = jnp.maximum(g, 0.0)
    out_ref[...] = jnp.dot(g, pW_ref[...],
                           preferred_element_type=jnp.float32) + pb_ref[...]


def kernel(x, label, seqs, edge_index, node_type, fcW, fcb, gcnW, gcnb,
           reW, re_wtype, re_b, predW, predb):
    f32 = jnp.float32
    src = edge_index[0].astype(jnp.int32)
    dst = edge_index[1].astype(jnp.int32)
    padlen = EPAD - E
    srcp = jnp.concatenate([src, jnp.full((padlen,), N, jnp.int32)])
    dstp = jnp.concatenate([dst, jnp.full((padlen,), N, jnp.int32)])
    src_f = srcp.reshape(TOTC, CHUNK)
    dst_f = dstp.reshape(TOTC, CHUNK)
    dst_w = dstp.reshape(NW, NCHUNK, CHUNK)
    x_pad = jnp.pad(x, ((0, NP - N), (0, 0)))
    ones128 = jnp.ones((CHUNK, D), f32)
    zacc = jnp.zeros((NP, D), f32)
    labf = jnp.broadcast_to(
        jnp.pad(label.astype(f32), (0, NP - N))[:, None], (NP, 8))
    ego = seqs[:, 0].astype(jnp.int32)
    ego_w = ego.reshape(NW, _EGO_PER_W)
    predW_pad = jnp.pad(predW, ((0, 0), (0, D - C)))
    predb_pad = jnp.pad(predb, (0, D - C)).reshape(1, D)

    deg2 = _sc_degree(dst_w, ones128, zacc)

    hs0 = pl.pallas_call(
        _tc_proj_body,
        out_shape=jax.ShapeDtypeStruct((NP, D), f32),
    )(x_pad, fcW, fcb.reshape(T, 1, D), gcnW[0], deg2)

    acc1 = _sc_segsum(hs0, src_f, dst_f, zacc)
    hs1, misc = pl.pallas_call(
        _tc_layer_body,
        out_shape=(jax.ShapeDtypeStruct((NP, D), f32),
                   jax.ShapeDtypeStruct((NP, D), f32)),
    )(acc1, hs0, deg2, gcnb[0].reshape(1, D), gcnW[1], labf)
    acc2 = _sc_segsum(hs1, src_f, dst_f, zacc)

    a0, a1, h1, mg = _sc_ego_gather(acc2[0], acc2[1], hs1, misc, ego_w)

    out = pl.pallas_call(
        _tc_final_body,
        out_shape=jax.ShapeDtypeStruct((B, D), f32),
    )(a0, a1, h1, mg, gcnb[1].reshape(1, D), predW_pad, predb_pad)

    return (out[:, :C], mg[:, 1].astype(label.dtype))
